# baseline (speedup 1.0000x reference)
import functools
import math

import jax
import jax.numpy as jnp
from jax import lax
from jax.experimental import pallas as pl
from jax.experimental.pallas import tpu as pltpu
from jax.experimental.pallas import tpu_sc as plsc

D_MODEL = 64
SCALE = math.sqrt(D_MODEL)
BW = 128

_INFO = plsc.get_sparse_core_info()
NC = _INFO.num_cores
NS = _INFO.num_subcores
NW = NC * NS


def _build(B0: int, S: int):
    n_bt = B0 // BW
    bt_per_w = n_bt // NW
    n_blocks = bt_per_w * S
    n_pair = n_blocks // 2

    mesh = plsc.VectorSubcoreMesh(core_axis_name="c", subcore_axis_name="s")

    @functools.partial(
        pl.kernel,
        mesh=mesh,
        out_type=jax.ShapeDtypeStruct((S, D_MODEL // 8, n_bt, 8, BW),
                                      jnp.float32),
        scratch_types=[
            pltpu.VMEM((bt_per_w, S, BW), jnp.int32),
            pltpu.VMEM((BW, D_MODEL), jnp.float32),
            pltpu.VMEM((BW, D_MODEL), jnp.float32),
            pltpu.VMEM((D_MODEL // 8, 8, BW + 1), jnp.float32),
            pltpu.VMEM((D_MODEL // 8, 8, BW + 1), jnp.float32),
            pltpu.SemaphoreType.DMA,
            pltpu.SemaphoreType.DMA,
            pltpu.SemaphoreType.DMA,
            pltpu.SemaphoreType.DMA,
        ],
        compiler_params=pltpu.CompilerParams(use_tc_tiling_on_sc=False, needs_layout_passes=False),
    )
    def emb(xt_hbm, table_hbm, out_hbm, idx_v, g0, g1, o0, o1,
            gsem0, gsem1, wsem0, wsem1):
        wid = lax.axis_index("s") * NC + lax.axis_index("c")
        bt0 = wid * bt_per_w
        g = (g0, g1)
        o = (o0, o1)
        gsem = (gsem0, gsem1)
        wsem = (wsem0, wsem1)

        for j in range(bt_per_w):
            pltpu.sync_copy(
                xt_hbm.at[:, pl.ds((bt0 + j) * BW, BW)], idx_v.at[j])

        iota16 = lax.iota(jnp.int32, 16)

        def fire(n, b):
            j = n // S
            s = n - j * S
            pltpu.async_copy(table_hbm.at[idx_v.at[j, s]], g[b], gsem[b])

        def gather_wait(b):
            pltpu.make_async_copy(
                table_hbm.at[pl.ds(0, BW)], g[b], gsem[b]).wait()

        def wb_start(n, b):
            j = n // S
            s = n - j * S
            pltpu.async_copy(
                o[b].at[:, :, pl.ds(0, BW)],
                out_hbm.at[s, pl.ds(0, D_MODEL // 8), bt0 + j],
                wsem[b])

        def wb_wait(b):
            pltpu.make_async_copy(
                o[b].at[:, :, pl.ds(0, BW)],
                out_hbm.at[0, pl.ds(0, D_MODEL // 8), 0],
                wsem[b]).wait()

        def transpose_scale(b):
            dt_vecs = [(q * 16 + iota16) // 8 for q in range(D_MODEL // 16)]
            dr_vecs = [(q * 16 + iota16) % 8 for q in range(D_MODEL // 16)]

            def body(k, carry):
                work = []
                for u in range(4):
                    bc = k * 4 + u
                    bcv = jnp.zeros((16,), jnp.int32) + bc
                    for q in range(D_MODEL // 16):
                        work.append(
                            (q, bcv, g[b][bc, pl.ds(q * 16, 16)] * SCALE))
                for q, bcv, vals in work:
                    plsc.store_scatter(
                        o[b], [dt_vecs[q], dr_vecs[q], bcv], vals)
                return carry
            lax.fori_loop(0, BW // 4, body, 0)

        fire(0, 0)

        def pair(p, carry):
            n0 = 2 * p

            @pl.when(p > 0)
            def _():
                wb_wait(0)
            fire(n0 + 1, 1)
            gather_wait(0)
            transpose_scale(0)
            wb_start(n0, 0)

            @pl.when(p > 0)
            def _():
                wb_wait(1)

            @pl.when(p < n_pair - 1)
            def _():
                fire(n0 + 2, 0)
            gather_wait(1)
            transpose_scale(1)
            wb_start(n0 + 1, 1)
            return carry

        lax.fori_loop(0, n_pair, pair, 0)
        wb_wait(0)
        wb_wait(1)

    return emb


def _build_format(V: int):
    n_ch = -(-V // BW)
    per_w = -(-n_ch // NW)
    per_w += per_w % 2
    n_pair = per_w // 2
    last0 = V - BW
    mesh = plsc.VectorSubcoreMesh(core_axis_name="c", subcore_axis_name="s")

    @functools.partial(
        pl.kernel,
        mesh=mesh,
        out_type=jax.ShapeDtypeStruct((V, D_MODEL), jnp.float32),
        scratch_types=[
            pltpu.VMEM((D_MODEL, BW), jnp.float32),
            pltpu.VMEM((D_MODEL, BW), jnp.float32),
            pltpu.VMEM((BW, D_MODEL + 1), jnp.float32),
            pltpu.VMEM((BW, D_MODEL + 1), jnp.float32),
            pltpu.SemaphoreType.DMA,
            pltpu.SemaphoreType.DMA,
            pltpu.SemaphoreType.DMA,
            pltpu.SemaphoreType.DMA,
        ],
        compiler_params=pltpu.CompilerParams(
            use_tc_tiling_on_sc=False, needs_layout_passes=False),
    )
    def fmt(tt_hbm, out_hbm, s0, s1, t0, t1, is0, is1, os0, os1):
        wid = lax.axis_index("s") * NC + lax.axis_index("c")
        slab = (s0, s1)
        tbuf = (t0, t1)
        isem = (is0, is1)
        osem = (os0, os1)
        iota16 = lax.iota(jnp.int32, 16)

        def start(t):
            return lax.min((t * NW + wid) * BW, last0)

        def fire(t, b):
            pltpu.async_copy(
                tt_hbm.at[:, pl.ds(start(t), BW)], slab[b], isem[b])

        def in_wait(b):
            pltpu.make_async_copy(
                tt_hbm.at[:, pl.ds(0, BW)], slab[b], isem[b]).wait()

        def out_start(t, b):
            pltpu.async_copy(
                tbuf[b].at[:, pl.ds(0, D_MODEL)],
                out_hbm.at[pl.ds(start(t), BW)], osem[b])

        def out_wait(b):
            pltpu.make_async_copy(
                tbuf[b].at[:, pl.ds(0, D_MODEL)],
                out_hbm.at[pl.ds(0, BW)], osem[b]).wait()

        def transpose(b):
            row_vecs = [q * 16 + iota16 for q in range(BW // 16)]

            def body(d, carry):
                dv = jnp.zeros((16,), jnp.int32) + d
                work = [(q, slab[b][d, pl.ds(q * 16, 16)])
                        for q in range(BW // 16)]
                for q, vals in work:
                    plsc.store_scatter(tbuf[b], [row_vecs[q], dv], vals)
                return carry
            lax.fori_loop(0, D_MODEL, body, 0)

        fire(0, 0)

        def pair(p, carry):
            t = 2 * p

            @pl.when(p > 0)
            def _():
                out_wait(0)
            fire(t + 1, 1)
            in_wait(0)
            transpose(0)
            out_start(t, 0)

            @pl.when(p > 0)
            def _():
                out_wait(1)

            @pl.when(p < n_pair - 1)
            def _():
                fire(t + 2, 0)
            in_wait(1)
            transpose(1)
            out_start(t + 1, 1)
            return carry

        lax.fori_loop(0, n_pair, pair, 0)
        out_wait(0)
        out_wait(1)

    return fmt


def _impl(x, table):
    B0, S = x.shape
    V = table.shape[0]
    traw = _build_format(V)(table.T)
    raw = _build(B0, S)(x.T, traw)
    out5 = raw.reshape(S, D_MODEL // 8, B0 // BW, 8, BW)
    return out5.transpose(2, 4, 0, 1, 3).reshape(B0, S, D_MODEL)


kernel = jax.jit(_impl)


# speedup vs baseline: 6.5938x; 6.5938x over previous
import functools
import math

import jax
import jax.numpy as jnp
from jax import lax
from jax.experimental import pallas as pl
from jax.experimental.pallas import tpu as pltpu
from jax.experimental.pallas import tpu_sc as plsc

D_MODEL = 64
SCALE = math.sqrt(D_MODEL)
BW = 128

_INFO = plsc.get_sparse_core_info()
NC = _INFO.num_cores
NS = _INFO.num_subcores
NW = NC * NS


def _build(B0: int, S: int):
    n_bt = B0 // BW
    bt_per_w = n_bt // NW
    n_blocks = bt_per_w * S
    n_pair = n_blocks // 2

    mesh = plsc.VectorSubcoreMesh(core_axis_name="c", subcore_axis_name="s")

    @functools.partial(
        pl.kernel,
        mesh=mesh,
        out_type=jax.ShapeDtypeStruct((S, D_MODEL // 8, n_bt, 8, BW),
                                      jnp.float32),
        scratch_types=[
            pltpu.VMEM((bt_per_w, S, BW), jnp.int32),
            pltpu.VMEM((BW, D_MODEL), jnp.float32),
            pltpu.VMEM((BW, D_MODEL), jnp.float32),
            pltpu.VMEM((D_MODEL // 8, 8, BW + 1), jnp.float32),
            pltpu.VMEM((D_MODEL // 8, 8, BW + 1), jnp.float32),
            pltpu.SemaphoreType.DMA,
            pltpu.SemaphoreType.DMA,
            pltpu.SemaphoreType.DMA,
            pltpu.SemaphoreType.DMA,
        ],
        compiler_params=pltpu.CompilerParams(use_tc_tiling_on_sc=False, needs_layout_passes=False),
    )
    def emb(xt_hbm, table_hbm, out_hbm, idx_v, g0, g1, o0, o1,
            gsem0, gsem1, wsem0, wsem1):
        wid = lax.axis_index("s") * NC + lax.axis_index("c")
        bt0 = wid * bt_per_w
        g = (g0, g1)
        o = (o0, o1)
        gsem = (gsem0, gsem1)
        wsem = (wsem0, wsem1)

        for j in range(bt_per_w):
            pltpu.sync_copy(
                xt_hbm.at[:, pl.ds((bt0 + j) * BW, BW)], idx_v.at[j])

        iota16 = lax.iota(jnp.int32, 16)

        def fire(n, b):
            j = n // S
            s = n - j * S
            pltpu.async_copy(table_hbm.at[idx_v.at[j, s]], g[b], gsem[b])

        def gather_wait(b):
            pltpu.make_async_copy(
                table_hbm.at[pl.ds(0, BW)], g[b], gsem[b]).wait()

        def wb_start(n, b):
            j = n // S
            s = n - j * S
            pltpu.async_copy(
                o[b].at[:, :, pl.ds(0, BW)],
                out_hbm.at[s, pl.ds(0, D_MODEL // 8), bt0 + j],
                wsem[b])

        def wb_wait(b):
            pltpu.make_async_copy(
                o[b].at[:, :, pl.ds(0, BW)],
                out_hbm.at[0, pl.ds(0, D_MODEL // 8), 0],
                wsem[b]).wait()

        def transpose_scale(b):
            dt_vecs = [(q * 16 + iota16) // 8 for q in range(D_MODEL // 16)]
            dr_vecs = [(q * 16 + iota16) % 8 for q in range(D_MODEL // 16)]

            def body(k, carry):
                work = []
                for u in range(4):
                    bc = k * 4 + u
                    bcv = jnp.zeros((16,), jnp.int32) + bc
                    for q in range(D_MODEL // 16):
                        work.append(
                            (q, bcv, g[b][bc, pl.ds(q * 16, 16)] * SCALE))
                for q, bcv, vals in work:
                    plsc.store_scatter(
                        o[b], [dt_vecs[q], dr_vecs[q], bcv], vals)
                return carry
            lax.fori_loop(0, BW // 4, body, 0)

        fire(0, 0)

        def pair(p, carry):
            n0 = 2 * p

            @pl.when(p > 0)
            def _():
                wb_wait(0)
            fire(n0 + 1, 1)
            gather_wait(0)
            transpose_scale(0)
            wb_start(n0, 0)

            @pl.when(p > 0)
            def _():
                wb_wait(1)

            @pl.when(p < n_pair - 1)
            def _():
                fire(n0 + 2, 0)
            gather_wait(1)
            transpose_scale(1)
            wb_start(n0 + 1, 1)
            return carry

        lax.fori_loop(0, n_pair, pair, 0)
        wb_wait(0)
        wb_wait(1)

    return emb


def _impl(x, table):
    B0, S = x.shape
    raw = _build(B0, S)(x.T, table)
    out5 = raw.reshape(S, D_MODEL // 8, B0 // BW, 8, BW)
    return out5.transpose(2, 4, 0, 1, 3).reshape(B0, S, D_MODEL)


kernel = jax.jit(_impl)
